# baseline (speedup 1.0000x reference)
import functools

import jax
import jax.numpy as jnp
from jax import lax
from jax.experimental import pallas as pl
from jax.experimental.pallas import tpu as pltpu
from jax.experimental.pallas import tpu_sc as plsc

NC = 2
NS = 16
L = 16
NW = NC * NS
B = 16384
EMB = 64
BPW = B // NW
CHUNK = 128
NCHUNK = BPW // CHUNK


def _mf_body(u_id_hbm, i_id_hbm, user_emb_hbm, user_bias_hbm, item_emb_hbm,
             item_bias_hbm, mean_hbm, out_hbm,
             idx_u, idx_i, urows, irows, bu, bi, out_v, mean_v, sem):
    wid = lax.axis_index("s") * NC + lax.axis_index("c")
    base = wid * BPW

    pltpu.sync_copy(mean_hbm, mean_v)
    for j in range(NCHUNK):
        pltpu.sync_copy(u_id_hbm.at[pl.ds(base + j * CHUNK, CHUNK)], idx_u.at[j])
        pltpu.sync_copy(i_id_hbm.at[pl.ds(base + j * CHUNK, CHUNK)], idx_i.at[j])

    handles = []
    for j in range(NCHUNK):
        s = pl.ds(j * CHUNK, CHUNK)
        handles.append(pltpu.async_copy(
            user_emb_hbm.at[idx_u.at[j]], urows.at[s], sem))
        handles.append(pltpu.async_copy(
            item_emb_hbm.at[idx_i.at[j]], irows.at[s], sem))
        handles.append(pltpu.async_copy(
            user_bias_hbm.at[idx_u.at[j]], bu.at[s], sem))
        handles.append(pltpu.async_copy(
            item_bias_hbm.at[idx_i.at[j]], bi.at[s], sem))
    for h in handles:
        h.wait()

    mean = mean_v[...]

    def blk_body(blk, carry):
        rowv = blk * L + lax.iota(jnp.int32, L)
        acc = jnp.zeros((L,), jnp.float32)
        for e in range(EMB):
            colv = jnp.full((L,), e, jnp.int32)
            uv = plsc.load_gather(urows, [rowv, colv])
            iv = plsc.load_gather(irows, [rowv, colv])
            acc = acc + uv * iv
        bu_v = bu[pl.ds(blk * L, L)]
        bi_v = bi[pl.ds(blk * L, L)]
        out_v[pl.ds(blk * L, L)] = acc + bu_v + bi_v + mean
        return carry

    lax.fori_loop(0, BPW // L, blk_body, 0)

    pltpu.sync_copy(out_v, out_hbm.at[pl.ds(base, BPW)])


@functools.cache
def _build():
    mesh = plsc.VectorSubcoreMesh(
        core_axis_name="c", subcore_axis_name="s",
        num_cores=NC, num_subcores=NS)
    return pl.kernel(
        _mf_body,
        out_type=jax.ShapeDtypeStruct((B,), jnp.float32),
        mesh=mesh,
        compiler_params=pltpu.CompilerParams(
            needs_layout_passes=False, use_tc_tiling_on_sc=False),
        scratch_types=[
            pltpu.VMEM((NCHUNK, CHUNK), jnp.int32),
            pltpu.VMEM((NCHUNK, CHUNK), jnp.int32),
            pltpu.VMEM((BPW, EMB), jnp.float32),
            pltpu.VMEM((BPW, EMB), jnp.float32),
            pltpu.VMEM((BPW,), jnp.float32),
            pltpu.VMEM((BPW,), jnp.float32),
            pltpu.VMEM((BPW,), jnp.float32),
            pltpu.VMEM((L,), jnp.float32),
            pltpu.SemaphoreType.DMA,
        ],
    )


def kernel(u_id, i_id, user_emb, user_bias, item_emb, item_bias, mean):
    mean16 = jnp.broadcast_to(mean.reshape(()), (L,))
    return _build()(u_id.astype(jnp.int32), i_id.astype(jnp.int32),
                    user_emb, user_bias.reshape(-1),
                    item_emb, item_bias.reshape(-1), mean16)


# speedup vs baseline: 1.8785x; 1.8785x over previous
import functools

import jax
import jax.numpy as jnp
from jax import lax
from jax.experimental import pallas as pl
from jax.experimental.pallas import tpu as pltpu
from jax.experimental.pallas import tpu_sc as plsc

NC = 2
NS = 16
L = 16
NW = NC * NS
B = 16384
EMB = 64
BPW = B // NW
CHUNK = 128
NCHUNK = BPW // CHUNK
GPC = CHUNK // L
SW = 128
RB = 128


def _gather_body(us_hbm, up_hbm, is_hbm, ip_hbm, uemb_hbm, iemb_hbm,
                 su_hbm, si_hbm,
                 idx_v, pos_v, strip_v, rowbuf, sem):
    wid = lax.axis_index("s") * NC + lax.axis_index("c")
    base = wid * BPW

    for sorted_hbm, posn_hbm, emb_hbm, scratch_hbm in (
            (us_hbm, up_hbm, uemb_hbm, su_hbm),
            (is_hbm, ip_hbm, iemb_hbm, si_hbm)):
        for j in range(NCHUNK):
            pltpu.sync_copy(sorted_hbm.at[pl.ds(base + j * CHUNK, CHUNK)],
                            idx_v.at[j])
            pltpu.sync_copy(posn_hbm.at[pl.ds(base + j * CHUNK, CHUNK)],
                            pos_v.at[j])

        cur = jnp.int32(-1)
        for ch in range(NCHUNK):
            def grp(g, cur, _ch=ch):
                ivec = idx_v[_ch, pl.ds(g * L, L)]
                for k in range(L):
                    u = ivec[k]
                    gs = u // SW
                    c = u - gs * SW

                    @pl.when(gs != cur)
                    def _():
                        pltpu.sync_copy(
                            emb_hbm.at[:, pl.ds(pl.multiple_of(gs * SW, SW),
                                                SW)],
                            strip_v)
                    cur = gs
                    cv = jnp.full((L,), c, jnp.int32)
                    for r in range(EMB // L):
                        ev = lax.iota(jnp.int32, L) + r * L
                        col = plsc.load_gather(strip_v, [ev, cv])
                        rowbuf[g * L + k, pl.ds(r * L, L)] = col
                return cur

            cur = lax.fori_loop(0, GPC, grp, cur)
            pltpu.async_copy(rowbuf, scratch_hbm.at[pos_v.at[ch]], sem).wait()


def _dot_body(su_hbm, si_hbm, u_id_hbm, i_id_hbm, ub_hbm, ib_hbm, mean_hbm,
              out_hbm,
              idx_u, idx_i, ublk, iblk, bu, bi, out_v, mean_v, sem):
    wid = lax.axis_index("s") * NC + lax.axis_index("c")
    base = wid * BPW

    pltpu.sync_copy(mean_hbm, mean_v)
    for j in range(NCHUNK):
        pltpu.sync_copy(u_id_hbm.at[pl.ds(base + j * CHUNK, CHUNK)],
                        idx_u.at[j])
        pltpu.sync_copy(i_id_hbm.at[pl.ds(base + j * CHUNK, CHUNK)],
                        idx_i.at[j])
    handles = []
    for j in range(NCHUNK):
        s = pl.ds(j * CHUNK, CHUNK)
        handles.append(pltpu.async_copy(ub_hbm.at[idx_u.at[j]], bu.at[s], sem))
        handles.append(pltpu.async_copy(ib_hbm.at[idx_i.at[j]], bi.at[s], sem))
    for h in handles:
        h.wait()

    mean = mean_v[...]
    for ch in range(NCHUNK):
        pltpu.sync_copy(su_hbm.at[pl.ds(base + ch * CHUNK, CHUNK), :], ublk)
        pltpu.sync_copy(si_hbm.at[pl.ds(base + ch * CHUNK, CHUNK), :], iblk)

        def blk(g, carry, _ch=ch):
            rowv = g * L + lax.iota(jnp.int32, L)
            acc = jnp.zeros((L,), jnp.float32)
            for e in range(EMB):
                colv = jnp.full((L,), e, jnp.int32)
                uv = plsc.load_gather(ublk, [rowv, colv])
                iv = plsc.load_gather(iblk, [rowv, colv])
                acc = acc + uv * iv
            p = _ch * CHUNK + g * L
            out_v[pl.ds(p, L)] = acc + bu[pl.ds(p, L)] + bi[pl.ds(p, L)] + mean
            return carry

        lax.fori_loop(0, GPC, blk, 0)

    pltpu.sync_copy(out_v, out_hbm.at[pl.ds(base, BPW)])


@functools.cache
def _build():
    mesh = plsc.VectorSubcoreMesh(
        core_axis_name="c", subcore_axis_name="s",
        num_cores=NC, num_subcores=NS)
    cp = pltpu.CompilerParams(
        needs_layout_passes=False, use_tc_tiling_on_sc=True)

    gather = pl.kernel(
        _gather_body,
        out_type=(jax.ShapeDtypeStruct((B, RB), jnp.float32),
                  jax.ShapeDtypeStruct((B, RB), jnp.float32)),
        mesh=mesh,
        compiler_params=cp,
        scratch_types=[
            pltpu.VMEM((NCHUNK, CHUNK), jnp.int32),
            pltpu.VMEM((NCHUNK, CHUNK), jnp.int32),
            pltpu.VMEM((EMB, SW), jnp.float32),
            pltpu.VMEM((CHUNK, RB), jnp.float32),
            pltpu.SemaphoreType.DMA,
        ],
    )
    dot = pl.kernel(
        _dot_body,
        out_type=jax.ShapeDtypeStruct((B,), jnp.float32),
        mesh=mesh,
        compiler_params=cp,
        scratch_types=[
            pltpu.VMEM((NCHUNK, CHUNK), jnp.int32),
            pltpu.VMEM((NCHUNK, CHUNK), jnp.int32),
            pltpu.VMEM((CHUNK, RB), jnp.float32),
            pltpu.VMEM((CHUNK, RB), jnp.float32),
            pltpu.VMEM((BPW,), jnp.float32),
            pltpu.VMEM((BPW,), jnp.float32),
            pltpu.VMEM((BPW,), jnp.float32),
            pltpu.VMEM((L,), jnp.float32),
            pltpu.SemaphoreType.DMA,
        ],
    )
    return gather, dot


def kernel(u_id, i_id, user_emb, user_bias, item_emb, item_bias, mean):
    gather, dot = _build()
    u_id = u_id.astype(jnp.int32)
    i_id = i_id.astype(jnp.int32)
    up = jnp.argsort(u_id).astype(jnp.int32)
    ip = jnp.argsort(i_id).astype(jnp.int32)
    us = u_id[up]
    isrt = i_id[ip]
    scratch_u, scratch_i = gather(us, up, isrt, ip, user_emb.T, item_emb.T)
    mean16 = jnp.broadcast_to(mean.reshape(()), (L,))
    return dot(scratch_u, scratch_i, u_id, i_id,
               user_bias.reshape(-1), item_bias.reshape(-1), mean16)
